# baseline (speedup 1.0000x reference)
import jax
import jax.numpy as jnp
from jax.experimental import pallas as pl

_BM = 512
_BK = 512


def _vq_kernel(zsq_ref, csq_ref, z_ref, tab_ref, out_ref):
    z = z_ref[...]
    zsq = zsq_ref[...]
    num_k = tab_ref.shape[0] // _BK

    def body(j, carry):
        best, bidx = carry
        tb = tab_ref[pl.ds(j * _BK, _BK), :]
        cross = jax.lax.dot_general(
            z, tb, (((1,), (1,)), ((), ())),
            preferred_element_type=jnp.float32)
        csq = csq_ref[0, pl.ds(j * _BK, _BK)]
        d = zsq - 2.0 * cross + csq[None, :]
        lmin = jnp.min(d, axis=1)
        larg = jnp.argmin(d, axis=1).astype(jnp.int32) + j * _BK
        better = lmin < best
        best = jnp.where(better, lmin, best)
        bidx = jnp.where(better, larg, bidx)
        return best, bidx

    init = (jnp.full((_BM,), jnp.inf, dtype=jnp.float32),
            jnp.zeros((_BM,), dtype=jnp.int32))
    _, bidx = jax.lax.fori_loop(0, num_k, body, init)
    out_ref[...] = bidx


def kernel(z_e_x, table):
    B, D = z_e_x.shape
    K, _ = table.shape
    z_sq = jnp.sum(z_e_x * z_e_x, axis=-1, keepdims=True)
    c_sq = jnp.sum(table * table, axis=-1)[None, :]
    return pl.pallas_call(
        _vq_kernel,
        grid=(B // _BM,),
        in_specs=[
            pl.BlockSpec((_BM, 1), lambda i: (i, 0)),
            pl.BlockSpec((1, K), lambda i: (0, 0)),
            pl.BlockSpec((_BM, D), lambda i: (i, 0)),
            pl.BlockSpec((K, D), lambda i: (0, 0)),
        ],
        out_specs=pl.BlockSpec((_BM,), lambda i: (i,)),
        out_shape=jax.ShapeDtypeStruct((B,), jnp.int32),
    )(z_sq, c_sq, z_e_x, table)


# speedup vs baseline: 2.9173x; 2.9173x over previous
import jax
import jax.numpy as jnp
from jax.experimental import pallas as pl

_BM = 512
_BK = 512


def _vq_kernel(zsq_ref, csq_ref, z_ref, tabt_ref, out_ref):
    z = z_ref[...]
    zsq = zsq_ref[...]
    K = tabt_ref.shape[1]
    num_k = K // _BK

    rmin = jnp.full((_BM, _BK), jnp.inf, dtype=jnp.float32)
    rarg = jnp.zeros((_BM, _BK), dtype=jnp.int32)
    for j in range(num_k):
        tb = tabt_ref[:, j * _BK:(j + 1) * _BK]
        cross = jnp.dot(z, tb, preferred_element_type=jnp.float32)
        csq = csq_ref[:, j * _BK:(j + 1) * _BK]
        d = zsq - 2.0 * cross + csq
        upd = d < rmin
        rmin = jnp.where(upd, d, rmin)
        rarg = jnp.where(upd, jnp.int32(j), rarg)

    m = jnp.min(rmin, axis=1, keepdims=True)
    lane = jax.lax.broadcasted_iota(jnp.int32, (_BM, _BK), 1)
    gidx = rarg * _BK + lane
    cand = jnp.where(rmin == m, gidx, jnp.int32(K))
    out_ref[...] = jnp.min(cand, axis=1)


def kernel(z_e_x, table):
    B, D = z_e_x.shape
    K, _ = table.shape
    z_sq = jnp.sum(z_e_x * z_e_x, axis=-1, keepdims=True)
    c_sq = jnp.sum(table * table, axis=-1)[None, :]
    table_t = table.T
    return pl.pallas_call(
        _vq_kernel,
        grid=(B // _BM,),
        in_specs=[
            pl.BlockSpec((_BM, 1), lambda i: (i, 0)),
            pl.BlockSpec((1, K), lambda i: (0, 0)),
            pl.BlockSpec((_BM, D), lambda i: (i, 0)),
            pl.BlockSpec((D, K), lambda i: (0, 0)),
        ],
        out_specs=pl.BlockSpec((_BM,), lambda i: (i,)),
        out_shape=jax.ShapeDtypeStruct((B,), jnp.int32),
    )(z_sq, c_sq, z_e_x, table_t)


# speedup vs baseline: 3.7793x; 1.2955x over previous
import jax
import jax.numpy as jnp
from jax.experimental import pallas as pl

_BM = 512
_BK = 512


def _vq_kernel(zsq_ref, csq_ref, z_ref, tab_ref, out_ref):
    z = z_ref[...]
    zsq = zsq_ref[...]
    K = tab_ref.shape[0]
    num_k = K // _BK

    rmin = jnp.full((_BM, _BK), jnp.inf, dtype=jnp.float32)
    rarg = jnp.zeros((_BM, _BK), dtype=jnp.int32)
    for j in range(num_k):
        tb = tab_ref[j * _BK:(j + 1) * _BK, :]
        cross = jax.lax.dot_general(
            z, tb, (((1,), (1,)), ((), ())),
            preferred_element_type=jnp.float32)
        csq = csq_ref[:, j * _BK:(j + 1) * _BK]
        d = zsq - 2.0 * cross + csq
        upd = d < rmin
        rmin = jnp.where(upd, d, rmin)
        rarg = jnp.where(upd, jnp.int32(j), rarg)

    m = jnp.min(rmin, axis=1, keepdims=True)
    lane = jax.lax.broadcasted_iota(jnp.int32, (_BM, _BK), 1)
    gidx = rarg * _BK + lane
    cand = jnp.where(rmin == m, gidx, jnp.int32(K))
    out_ref[...] = jnp.min(cand, axis=1)


def kernel(z_e_x, table):
    B, D = z_e_x.shape
    K, _ = table.shape
    z_sq = jnp.sum(z_e_x * z_e_x, axis=-1, keepdims=True)
    c_sq = jnp.sum(table * table, axis=-1)[None, :]
    return pl.pallas_call(
        _vq_kernel,
        grid=(B // _BM,),
        in_specs=[
            pl.BlockSpec((_BM, 1), lambda i: (i, 0)),
            pl.BlockSpec((1, K), lambda i: (0, 0)),
            pl.BlockSpec((_BM, D), lambda i: (i, 0)),
            pl.BlockSpec((K, D), lambda i: (0, 0)),
        ],
        out_specs=pl.BlockSpec((_BM,), lambda i: (i,)),
        out_shape=jax.ShapeDtypeStruct((B,), jnp.int32),
    )(z_sq, c_sq, z_e_x, table)


# speedup vs baseline: 4.6356x; 1.2266x over previous
import jax
import jax.numpy as jnp
from jax.experimental import pallas as pl

_BM = 512
_BK = 512


def _vq_kernel(zsq_ref, csq_ref, z_ref, tab_ref, out_ref):
    z = z_ref[...]
    zsq = zsq_ref[...]
    K = tab_ref.shape[0]
    num_k = K // _BK

    nq = _BK // 128
    rmin = jnp.full((_BM, 128), jnp.inf, dtype=jnp.float32)
    rpk = jnp.zeros((_BM, 128), dtype=jnp.int32)
    for j in range(num_k):
        tb = tab_ref[j * _BK:(j + 1) * _BK, :]
        cross = jax.lax.dot_general(
            z, tb, (((1,), (1,)), ((), ())),
            preferred_element_type=jnp.float32)
        csq = csq_ref[:, j * _BK:(j + 1) * _BK]
        d = zsq - 2.0 * cross + csq
        for q in range(nq):
            dq = d[:, q * 128:(q + 1) * 128]
            upd = dq < rmin
            rmin = jnp.where(upd, dq, rmin)
            rpk = jnp.where(upd, jnp.int32(j * nq + q), rpk)

    m = jnp.min(rmin, axis=1, keepdims=True)
    lane = jax.lax.broadcasted_iota(jnp.int32, (_BM, 128), 1)
    gidx = rpk * 128 + lane
    cand = jnp.where(rmin == m, gidx, jnp.int32(K))
    out_ref[...] = jnp.min(cand, axis=1)


def kernel(z_e_x, table):
    B, D = z_e_x.shape
    K, _ = table.shape
    z_sq = jnp.sum(z_e_x * z_e_x, axis=-1, keepdims=True)
    c_sq = jnp.sum(table * table, axis=-1)[None, :]
    return pl.pallas_call(
        _vq_kernel,
        grid=(B // _BM,),
        in_specs=[
            pl.BlockSpec((_BM, 1), lambda i: (i, 0)),
            pl.BlockSpec((1, K), lambda i: (0, 0)),
            pl.BlockSpec((_BM, D), lambda i: (i, 0)),
            pl.BlockSpec((K, D), lambda i: (0, 0)),
        ],
        out_specs=pl.BlockSpec((_BM,), lambda i: (i,)),
        out_shape=jax.ShapeDtypeStruct((B,), jnp.int32),
    )(z_sq, c_sq, z_e_x, table)


# speedup vs baseline: 5.9120x; 1.2753x over previous
import jax
import jax.numpy as jnp
from jax.experimental import pallas as pl

_BM = 512
_BK = 512
_NS = 32


def _vq_kernel(zsqt_ref, csqt_ref, z_ref, tab_ref, out_ref):
    z = z_ref[...]
    zsqt = zsqt_ref[...]
    K = tab_ref.shape[0]
    num_k = K // _BK
    np_ = _BK // _NS

    rmin = jnp.full((_NS, _BM), jnp.inf, dtype=jnp.float32)
    rpk = jnp.zeros((_NS, _BM), dtype=jnp.int32)
    for j in range(num_k):
        tb = tab_ref[j * _BK:(j + 1) * _BK, :]
        crosst = jax.lax.dot_general(
            tb, z, (((1,), (1,)), ((), ())),
            preferred_element_type=jnp.float32)
        csq = csqt_ref[j * _BK:(j + 1) * _BK, :]
        dt = zsqt - 2.0 * crosst + csq
        d3 = dt.reshape(np_, _NS, _BM)
        for p in range(np_):
            dq = d3[p]
            upd = dq < rmin
            rmin = jnp.where(upd, dq, rmin)
            rpk = jnp.where(upd, jnp.int32(j * np_ + p), rpk)

    sio = jax.lax.broadcasted_iota(jnp.int32, (_NS, _BM), 0)
    v, k = rmin, rpk * _NS + sio
    s = _NS
    while s > 1:
        s //= 2
        va, vb = v[:s, :], v[s:, :]
        ka, kb = k[:s, :], k[s:, :]
        take_b = (vb < va) | ((vb == va) & (kb < ka))
        v = jnp.where(take_b, vb, va)
        k = jnp.where(take_b, kb, ka)
    out_ref[...] = k.reshape(_BM)


def kernel(z_e_x, table):
    B, D = z_e_x.shape
    K, _ = table.shape
    z_sq_t = jnp.sum(z_e_x * z_e_x, axis=-1)[None, :]
    c_sq_t = jnp.sum(table * table, axis=-1)[:, None]
    return pl.pallas_call(
        _vq_kernel,
        grid=(B // _BM,),
        in_specs=[
            pl.BlockSpec((1, _BM), lambda i: (0, i)),
            pl.BlockSpec((K, 1), lambda i: (0, 0)),
            pl.BlockSpec((_BM, D), lambda i: (i, 0)),
            pl.BlockSpec((K, D), lambda i: (0, 0)),
        ],
        out_specs=pl.BlockSpec((_BM,), lambda i: (i,)),
        out_shape=jax.ShapeDtypeStruct((B,), jnp.int32),
    )(z_sq_t, c_sq_t, z_e_x, table)
